# baseline (speedup 1.0000x reference)
import functools

import jax
import jax.numpy as jnp
from jax.experimental import pallas as pl
from jax.experimental.pallas import tpu as pltpu

N_C = 8192
N_L = 4096
D = 256
HALF = N_L // 2
BLK = 1024
NB = N_C // BLK
BLK_B = 256
NB_B = N_C // BLK_B
_F32 = jnp.float32


def _mm(a, b):
    return jax.lax.dot_general(a, b, (((1,), (0,)), ((), ())),
                               preferred_element_type=_F32)


def _relu(x):
    return jnp.maximum(x, 0.0)


def _full(shape):
    return pl.BlockSpec(shape, lambda j: tuple(0 for _ in shape))


def _row(b):
    return b.reshape(1, -1)


def _fwd_body(first_hop, G_ref, L_ref, w0T_ref, b0_ref, w1T_ref, b1_ref,
              w2T_ref, b2_ref, C_ref):
    if first_hop:
        Lv = jnp.broadcast_to(L_ref[...], (N_L, D))
    else:
        Lv = L_ref[...]
    Lf = jnp.concatenate([Lv[HALF:], Lv[:HALF]], axis=0)
    P = jnp.concatenate([Lv, Lf], axis=1)
    cm = _mm(G_ref[...], P)
    c1 = _relu(_mm(cm, w0T_ref[...]) + b0_ref[...])
    c2 = _relu(_mm(c1, w1T_ref[...]) + b1_ref[...])
    C_ref[...] = _mm(c2, w2T_ref[...]) + b2_ref[...]


def _run_fwd(first_hop, G, L, cw0T, cb0, cw1T, cb1, cw2T, cb2):
    return pl.pallas_call(
        functools.partial(_fwd_body, first_hop),
        grid=(NB,),
        in_specs=[pl.BlockSpec((BLK, N_L), lambda j: (j, 0)),
                  _full(L.shape), _full(cw0T.shape), _full(cb0.shape),
                  _full(cw1T.shape), _full(cb1.shape),
                  _full(cw2T.shape), _full(cb2.shape)],
        out_specs=pl.BlockSpec((BLK, D), lambda j: (j, 0)),
        out_shape=jax.ShapeDtypeStruct((N_C, D), _F32),
        compiler_params=pltpu.CompilerParams(
            dimension_semantics=("arbitrary",)),
    )(G, L, cw0T, cb0, cw1T, cb1, cw2T, cb2)


def _norm_body(C_ref, Cn_ref):
    C = C_ref[...]
    mu = jnp.mean(C, axis=0, keepdims=True)
    Cc = C - mu
    sd = jnp.sqrt(jnp.sum(Cc * Cc, axis=0, keepdims=True) / (N_C - 1))
    Cn_ref[...] = Cc / (sd + 1e-10)


def _run_norm(C):
    return pl.pallas_call(
        _norm_body,
        out_shape=jax.ShapeDtypeStruct((N_C, D), _F32))(C)


def _bwd_body(G_ref, Cn_ref, S_ref):
    j = pl.program_id(0)
    dS = jax.lax.dot_general(G_ref[...], Cn_ref[...],
                             (((0,), (0,)), ((), ())),
                             preferred_element_type=_F32)

    @pl.when(j == 0)
    def _():
        S_ref[...] = dS

    @pl.when(j > 0)
    def _():
        S_ref[...] += dS


def _run_bwd(G, Cn):
    return pl.pallas_call(
        _bwd_body,
        grid=(NB_B,),
        in_specs=[pl.BlockSpec((BLK_B, N_L), lambda j: (j, 0)),
                  pl.BlockSpec((BLK_B, D), lambda j: (j, 0))],
        out_specs=_full((N_L, D)),
        out_shape=jax.ShapeDtypeStruct((N_L, D), _F32),
        compiler_params=pltpu.CompilerParams(
            dimension_semantics=("arbitrary",)),
    )(G, Cn)


def _lit_body(first_hop, last_hop, S_ref, Lp_ref,
              lw0T_ref, lb0_ref, lw1T_ref, lb1_ref, lw2T_ref, lb2_ref,
              lnw_ref, lnb_ref,
              sw0T_ref, sb0_ref, sw1T_ref, sb1_ref, sw2T_ref, sb2_ref,
              out_ref):
    h = _relu(_mm(S_ref[...], lw0T_ref[...]) + lb0_ref[...])
    h = _relu(_mm(h, lw1T_ref[...]) + lb1_ref[...])
    h = _mm(h, lw2T_ref[...]) + lb2_ref[...]
    if first_hop:
        Lp = jnp.broadcast_to(Lp_ref[...], (N_L, D))
    else:
        Lp = Lp_ref[...]
    L = h + 0.1 * Lp

    m = jnp.mean(L, axis=1, keepdims=True)
    v = jnp.mean((L - m) ** 2, axis=1, keepdims=True)
    L = (L - m) / jnp.sqrt(v + 1e-5) * lnw_ref[...] + lnb_ref[...]

    if last_hop:
        V = jnp.concatenate([L[:HALF, :], L[HALF:, :]], axis=1)
        s = _relu(_mm(V, sw0T_ref[...]) + sb0_ref[...])
        s = _relu(_mm(s, sw1T_ref[...]) + sb1_ref[...])
        out_ref[...] = _mm(s, sw2T_ref[...]) + sb2_ref[...]
    else:
        out_ref[...] = L


def _run_lit(first_hop, last_hop, S, Lp, lw0T, lb0, lw1T, lb1, lw2T, lb2,
             lnw, lnb, sw0T, sb0, sw1T, sb1, sw2T, sb2):
    if last_hop:
        out_shape = jax.ShapeDtypeStruct((HALF, 1), _F32)
    else:
        out_shape = jax.ShapeDtypeStruct((N_L, D), _F32)
    return pl.pallas_call(
        functools.partial(_lit_body, first_hop, last_hop),
        out_shape=out_shape,
    )(S, Lp, lw0T, lb0, lw1T, lb1, lw2T, lb2, lnw, lnb,
      sw0T, sb0, sw1T, sb1, sw2T, sb2)


def kernel(G, L_init, CW0, Cb0, CW1, Cb1, CW2, Cb2, LW0, Lb0, LW1, Lb1,
           LW2, Lb2, SW0, Sb0, SW1, Sb1, SW2, Sb2, ln_w, ln_b):
    cw = (CW0.T, _row(Cb0), CW1.T, _row(Cb1), CW2.T, _row(Cb2))
    lw = (LW0.T, _row(Lb0), LW1.T, _row(Lb1), LW2.T, _row(Lb2))
    sw = (SW0.T, _row(Sb0), SW1.T, _row(Sb1), SW2.T, _row(Sb2))
    lnw, lnb = _row(ln_w), _row(ln_b)

    L = L_init
    for hop in range(3):
        first = hop == 0
        last = hop == 2
        C3 = _run_fwd(first, G, L, *cw)
        Cn = _run_norm(C3)
        S = _run_bwd(G, Cn)
        L = _run_lit(first, last, S, L, *lw, lnw, lnb, *sw)
    return L
